# Initial kernel scaffold; baseline (speedup 1.0000x reference)
import jax
import jax.numpy as jnp
from jax.experimental import pallas as pl


def kernel(x, emb):
    raise NotImplementedError("write your pallas kernel here")


# speedup vs baseline: 1.4589x; 1.4589x over previous
import functools

import jax
import jax.numpy as jnp
from jax import lax
from jax.experimental import pallas as pl
from jax.experimental.pallas import tpu as pltpu
from jax.experimental.pallas import tpu_sc as plsc

CHUNK = 1024


@functools.cache
def _make_gather(n, d):
    info = plsc.get_sparse_core_info()
    nc, ns = info.num_cores, info.num_subcores
    nw = nc * ns
    b_per_w = n // nw
    n_chunks = b_per_w // CHUNK
    assert b_per_w * nw == n and n_chunks * CHUNK == b_per_w
    mesh = plsc.VectorSubcoreMesh(core_axis_name="c", subcore_axis_name="s")

    @functools.partial(
        pl.kernel,
        mesh=mesh,
        out_type=jax.ShapeDtypeStruct((n, d), jnp.float32),
        compiler_params=pltpu.CompilerParams(use_tc_tiling_on_sc=False),
        scratch_types=[
            pltpu.VMEM((CHUNK,), jnp.int32),
            pltpu.VMEM((CHUNK, d), jnp.float32),
            pltpu.SemaphoreType.DMA,
        ],
    )
    def gather(table_hbm, idx_hbm, out_hbm, idx_v, rows_v, sem):
        wid = lax.axis_index("s") * nc + lax.axis_index("c")
        base = wid * b_per_w

        def body(c, carry):
            off = base + c * CHUNK
            pltpu.sync_copy(idx_hbm.at[pl.ds(off, CHUNK)], idx_v)
            pltpu.async_copy(table_hbm.at[idx_v], rows_v, sem).wait()
            pltpu.sync_copy(rows_v, out_hbm.at[pl.ds(off, CHUNK)])
            return carry

        lax.fori_loop(0, n_chunks, body, 0)

    return gather


def kernel(x, emb):
    bsz, hist = x.shape
    d = emb.shape[1]
    idx = x.reshape(-1).astype(jnp.int32)
    out = _make_gather(idx.shape[0], d)(emb, idx)
    return out.reshape(bsz, hist, d)


# speedup vs baseline: 1.5006x; 1.0286x over previous
import functools

import jax
import jax.numpy as jnp
from jax import lax
from jax.experimental import pallas as pl
from jax.experimental.pallas import tpu as pltpu
from jax.experimental.pallas import tpu_sc as plsc

CHUNK = 640
NBUF = 4
LOOKAHEAD = 2


@functools.cache
def _make_gather(n, d):
    info = plsc.get_sparse_core_info()
    nc, ns = info.num_cores, info.num_subcores
    nw = nc * ns
    b_per_w = n // nw
    n_chunks = b_per_w // CHUNK
    groups = n_chunks // NBUF
    assert b_per_w * nw == n and n_chunks * CHUNK == b_per_w
    assert groups * NBUF == n_chunks and groups >= 3
    mesh = plsc.VectorSubcoreMesh(core_axis_name="c", subcore_axis_name="s")

    @functools.partial(
        pl.kernel,
        mesh=mesh,
        out_type=jax.ShapeDtypeStruct((nw * b_per_w, d), jnp.float32),
        compiler_params=pltpu.CompilerParams(use_tc_tiling_on_sc=False),
        scratch_types=(
            [pltpu.VMEM((n_chunks, CHUNK), jnp.int32)]
            + [pltpu.VMEM((CHUNK, d), jnp.float32) for _ in range(NBUF)]
            + [pltpu.SemaphoreType.DMA for _ in range(2 * NBUF)]
        ),
    )
    def gather(table_hbm, idx_hbm, out_hbm, idx_v, *scratch):
        rbufs = scratch[:NBUF]
        gsems = scratch[NBUF:2 * NBUF]
        ssems = scratch[2 * NBUF:]
        wid = lax.axis_index("s") * nc + lax.axis_index("c")
        base = wid * b_per_w

        pltpu.sync_copy(idx_hbm.at[pl.ds(wid * n_chunks, n_chunks)], idx_v)

        def g_copy(c, b):
            return pltpu.make_async_copy(
                table_hbm.at[idx_v.at[c]], rbufs[b], gsems[b])

        def s_copy(c, b):
            return pltpu.make_async_copy(
                rbufs[b], out_hbm.at[pl.ds(base + c * CHUNK, CHUNK)], ssems[b])

        for c in range(LOOKAHEAD):
            g_copy(c, c % NBUF).start()

        for b in range(NBUF):
            c = b
            if c + LOOKAHEAD >= NBUF:
                s_copy(c - LOOKAHEAD, (c + LOOKAHEAD) % NBUF).wait()
            g_copy(c + LOOKAHEAD, (c + LOOKAHEAD) % NBUF).start()
            g_copy(c, b).wait()
            s_copy(c, b).start()

        def body(g, carry):
            for b in range(NBUF):
                c = g * NBUF + b
                bg = (b + LOOKAHEAD) % NBUF
                s_copy(c - LOOKAHEAD, bg).wait()
                g_copy(c + LOOKAHEAD, bg).start()
                g_copy(c, b).wait()
                s_copy(c, b).start()
            return carry

        lax.fori_loop(1, groups - 1, body, 0)

        for b in range(NBUF):
            c = (groups - 1) * NBUF + b
            if c + LOOKAHEAD < n_chunks:
                bg = (b + LOOKAHEAD) % NBUF
                s_copy(c - LOOKAHEAD, bg).wait()
                g_copy(c + LOOKAHEAD, bg).start()
            g_copy(c, b).wait()
            s_copy(c, b).start()

        for b in range(NBUF):
            c = n_chunks - NBUF + b
            s_copy(c, b).wait()

    return gather


def kernel(x, emb):
    bsz, hist = x.shape
    d = emb.shape[1]
    n = x.size
    idx = x.reshape(n // CHUNK, CHUNK).astype(jnp.int32)
    out = _make_gather(n, d)(emb, idx)
    return out.reshape(bsz, hist, d)


# speedup vs baseline: 1.5755x; 1.0499x over previous
import functools

import jax
import jax.numpy as jnp
from jax import lax
from jax.experimental import pallas as pl
from jax.experimental.pallas import tpu as pltpu
from jax.experimental.pallas import tpu_sc as plsc

CHUNK = 640
NBUF = 4
LOOKAHEAD = 2


@functools.cache
def _make_gather(n, d):
    info = plsc.get_sparse_core_info()
    nc, ns = info.num_cores, info.num_subcores
    nw = nc * ns
    b_per_w = n // nw
    n_chunks = b_per_w // CHUNK
    groups = n_chunks // NBUF
    assert b_per_w * nw == n and n_chunks * CHUNK == b_per_w
    assert groups * NBUF == n_chunks and groups >= 3
    mesh = plsc.VectorSubcoreMesh(core_axis_name="c", subcore_axis_name="s")

    @functools.partial(
        pl.kernel,
        mesh=mesh,
        out_type=jax.ShapeDtypeStruct((nw * b_per_w, d), jnp.float32),
        compiler_params=pltpu.CompilerParams(use_tc_tiling_on_sc=False),
        scratch_types=(
            [pltpu.VMEM((n_chunks, CHUNK), jnp.int32)]
            + [pltpu.VMEM((CHUNK, d), jnp.float32) for _ in range(NBUF)]
            + [pltpu.SemaphoreType.DMA for _ in range(2 * NBUF)]
        ),
    )
    def gather(table_hbm, idx_hbm, out_hbm, idx_v, *scratch):
        rbufs = scratch[:NBUF]
        gsems = scratch[NBUF:2 * NBUF]
        ssems = scratch[2 * NBUF:]
        wid = lax.axis_index("s") * nc + lax.axis_index("c")
        base = wid * b_per_w

        pltpu.sync_copy(idx_hbm.at[pl.ds(wid * n_chunks, n_chunks)], idx_v)

        def g_copy(c, b):
            return pltpu.make_async_copy(
                table_hbm.at[idx_v.at[c]], rbufs[b], gsems[b])

        def s_copy(c, b):
            return pltpu.make_async_copy(
                rbufs[b], out_hbm.at[pl.ds(base + c * CHUNK, CHUNK)], ssems[b])

        for c in range(LOOKAHEAD):
            g_copy(c, c % NBUF).start()

        for b in range(NBUF):
            c = b
            if c + LOOKAHEAD >= NBUF:
                s_copy(c - LOOKAHEAD, (c + LOOKAHEAD) % NBUF).wait()
            g_copy(c + LOOKAHEAD, (c + LOOKAHEAD) % NBUF).start()
            g_copy(c, b).wait()
            s_copy(c, b).start()

        def body(g, carry):
            for b in range(NBUF):
                c = g * NBUF + b
                bg = (b + LOOKAHEAD) % NBUF
                s_copy(c - LOOKAHEAD, bg).wait()
                g_copy(c + LOOKAHEAD, bg).start()
                g_copy(c, b).wait()
                s_copy(c, b).start()
            return carry

        lax.fori_loop(1, groups - 1, body, 0)

        for b in range(NBUF):
            c = (groups - 1) * NBUF + b
            if c + LOOKAHEAD < n_chunks:
                bg = (b + LOOKAHEAD) % NBUF
                s_copy(c - LOOKAHEAD, bg).wait()
                g_copy(c + LOOKAHEAD, bg).start()
            g_copy(c, b).wait()
            s_copy(c, b).start()

        for b in range(NBUF):
            c = n_chunks - NBUF + b
            s_copy(c, b).wait()

    return gather


def kernel(x, emb):
    bsz, hist = x.shape
    d = emb.shape[1]
    n = x.size
    idx = x.T.reshape(n // CHUNK, CHUNK).astype(jnp.int32)
    out = _make_gather(n, d)(emb, idx)
    return out.reshape(hist, bsz, d).transpose(1, 0, 2)
